# Initial kernel scaffold; baseline (speedup 1.0000x reference)
import jax
import jax.numpy as jnp
from jax.experimental import pallas as pl


def kernel(x, edge_index, Wl1, bl1, Wr1, g1, b1, Wl2, bl2, Wr2, g2, b2, Wd, bd):
    raise NotImplementedError("write your pallas kernel here")


# speedup vs baseline: 1.7244x; 1.7244x over previous
import functools

import jax
import jax.numpy as jnp
from jax import lax
from jax.experimental import pallas as pl
from jax.experimental.pallas import tpu as pltpu
from jax.experimental.pallas import tpu_sc as plsc

_CHUNK = 80


def _round_up(v, m):
    return (v + m - 1) // m * m


def _make_sc_agg(n_rows, n_edges, dim, with_deg):
    info = plsc.get_sparse_core_info()
    nc, ns = info.num_cores, info.num_subcores
    nw = nc * ns
    epw = n_edges // nw
    steps = epw // _CHUNK
    rps = n_rows // ns

    mesh = plsc.VectorSubcoreMesh(core_axis_name="c", subcore_axis_name="s")

    out_type = jax.ShapeDtypeStruct((nc * n_rows, dim), jnp.float32)
    scratch = [
        pltpu.VMEM_SHARED((n_rows, dim), jnp.float32),
        pltpu.VMEM((_CHUNK,), jnp.int32),
        pltpu.VMEM((_CHUNK,), jnp.int32),
        pltpu.VMEM((_CHUNK, dim), jnp.float32),
        pltpu.SemaphoreType.DMA,
    ]
    if with_deg:
        scratch += [
            pltpu.VMEM((_CHUNK,), jnp.int32),
            pltpu.VMEM((_CHUNK,), jnp.int32),
            pltpu.VMEM((_CHUNK, dim), jnp.float32),
        ]

    if with_deg:
        @functools.partial(
            pl.kernel, mesh=mesh, out_type=[out_type], scratch_types=scratch)
        def sc_kernel(feat, src, dst, onehot, dstm, dstdeg, zeros_f,
                      out_agg,
                      agg_sh, src_v, dst_v, rows_v, sem,
                      dstm_v, dstdeg_v, pay_v):
            c = lax.axis_index("c")
            s = lax.axis_index("s")
            wid = s * nc + c
            r0 = s * rps
            pltpu.sync_copy(zeros_f.at[pl.ds(r0, rps)],
                            agg_sh.at[pl.ds(r0, rps)])
            plsc.subcore_barrier()

            base = wid * epw

            def step(i, carry):
                off = base + i * _CHUNK
                pltpu.sync_copy(src.at[pl.ds(off, _CHUNK)], src_v)
                pltpu.sync_copy(dst.at[pl.ds(off, _CHUNK)], dst_v)
                pltpu.sync_copy(dstm.at[pl.ds(off, _CHUNK)], dstm_v)
                pltpu.sync_copy(dstdeg.at[pl.ds(off, _CHUNK)], dstdeg_v)
                pltpu.async_copy(feat.at[src_v], rows_v, sem).wait()
                pltpu.async_copy(onehot.at[dstm_v], pay_v, sem).wait()
                pltpu.sync_copy(rows_v, agg_sh.at[dst_v], add=True)
                pltpu.sync_copy(pay_v, agg_sh.at[dstdeg_v], add=True)
                return carry

            lax.fori_loop(0, steps, step, 0)
            plsc.subcore_barrier()
            o0 = c * n_rows + r0
            pltpu.sync_copy(agg_sh.at[pl.ds(r0, rps)],
                            out_agg.at[pl.ds(o0, rps)])
    else:
        @functools.partial(
            pl.kernel, mesh=mesh, out_type=[out_type], scratch_types=scratch)
        def sc_kernel(feat, src, dst, zeros_f,
                      out_agg,
                      agg_sh, src_v, dst_v, rows_v, sem):
            c = lax.axis_index("c")
            s = lax.axis_index("s")
            wid = s * nc + c
            r0 = s * rps
            pltpu.sync_copy(zeros_f.at[pl.ds(r0, rps)],
                            agg_sh.at[pl.ds(r0, rps)])
            plsc.subcore_barrier()

            base = wid * epw

            def step(i, carry):
                off = base + i * _CHUNK
                pltpu.sync_copy(src.at[pl.ds(off, _CHUNK)], src_v)
                pltpu.sync_copy(dst.at[pl.ds(off, _CHUNK)], dst_v)
                pltpu.async_copy(feat.at[src_v], rows_v, sem).wait()
                pltpu.sync_copy(rows_v, agg_sh.at[dst_v], add=True)
                return carry

            lax.fori_loop(0, steps, step, 0)
            plsc.subcore_barrier()
            o0 = c * n_rows + r0
            pltpu.sync_copy(agg_sh.at[pl.ds(r0, rps)],
                            out_agg.at[pl.ds(o0, rps)])

    return sc_kernel


def _tc_layer1(aggp_ref, degp0_ref, degp1_ref, x_ref, wl_ref, bl_ref,
               wr_ref, g_ref, b_ref, h_ref):
    n = x_ref.shape[0]
    np_ = aggp_ref.shape[0] // 2
    agg = aggp_ref[0:n, :] + aggp_ref[np_:np_ + n, :]
    deg = degp0_ref[0:n, 0:1] + degp1_ref[0:n, 0:1]
    mean = agg / jnp.maximum(deg, 1.0)
    h = (jnp.dot(mean, wl_ref[...], preferred_element_type=jnp.float32)
         + jnp.dot(x_ref[...], wr_ref[...], preferred_element_type=jnp.float32)
         + bl_ref[...])
    mu = jnp.mean(h, axis=0, keepdims=True)
    var = jnp.mean((h - mu) * (h - mu), axis=0, keepdims=True)
    hn = (h - mu) * lax.rsqrt(var + 1e-5) * g_ref[...] + b_ref[...]
    h_ref[...] = jnp.maximum(hn, 0.0)


def _tc_layer2(aggp_ref, degp0_ref, degp1_ref, x_ref, wl_ref, bl_ref,
               wr_ref, g_ref, b_ref, wd_ref, bd_ref, logits_ref, h_ref):
    n = x_ref.shape[0]
    np_ = aggp_ref.shape[0] // 2
    agg = aggp_ref[0:n, :] + aggp_ref[np_:np_ + n, :]
    deg = degp0_ref[0:n, 0:1] + degp1_ref[0:n, 0:1]
    mean = agg / jnp.maximum(deg, 1.0)
    h = (jnp.dot(mean, wl_ref[...], preferred_element_type=jnp.float32)
         + jnp.dot(x_ref[...], wr_ref[...], preferred_element_type=jnp.float32)
         + bl_ref[...])
    mu = jnp.mean(h, axis=0, keepdims=True)
    var = jnp.mean((h - mu) * (h - mu), axis=0, keepdims=True)
    hn = (h - mu) * lax.rsqrt(var + 1e-5) * g_ref[...] + b_ref[...]
    h2 = jnp.maximum(hn, 0.0)
    h_ref[...] = h2
    logits_ref[...] = (
        jnp.dot(h2, wd_ref[...], preferred_element_type=jnp.float32)
        + bd_ref[...])


def kernel(x, edge_index, Wl1, bl1, Wr1, g1, b1, Wl2, bl2, Wr2, g2, b2,
           Wd, bd):
    n, d = x.shape
    e = edge_index.shape[1]
    d_out = Wd.shape[0]

    n_pad = _round_up(n, 16 * 8)
    n_deg = n_pad // 8
    n_l1 = _round_up(n_pad + n_deg, 16 * 8)

    src = edge_index[0]
    dst = edge_index[1]
    dstm = dst % 8
    dstdeg = n_pad + dst // 8
    onehot = (jnp.arange(d, dtype=jnp.int32)[None, :] // 16
              == jnp.arange(8, dtype=jnp.int32)[:, None]).astype(jnp.float32)
    zeros_l1 = jnp.zeros((n_l1, d), jnp.float32)
    zeros_l2 = jnp.zeros((n_pad, d), jnp.float32)

    sc_l1 = _make_sc_agg(n_l1, e, d, with_deg=True)
    sc_l2 = _make_sc_agg(n_pad, e, d, with_deg=False)

    out1, = sc_l1(x, src, dst, onehot, dstm, dstdeg, zeros_l1)
    degp0 = out1[n_pad:n_pad + n_deg].reshape(n_pad, 16)
    degp1 = out1[n_l1 + n_pad:n_l1 + n_pad + n_deg].reshape(n_pad, 16)

    h1 = pl.pallas_call(
        _tc_layer1,
        out_shape=jax.ShapeDtypeStruct((n, d), jnp.float32),
    )(out1, degp0, degp1, x, Wl1.T, bl1.reshape(1, d), Wr1.T,
      g1.reshape(1, d), b1.reshape(1, d))

    aggp2, = sc_l2(h1, src, dst, zeros_l2)

    logits, h2 = pl.pallas_call(
        _tc_layer2,
        out_shape=[
            jax.ShapeDtypeStruct((n, d_out), jnp.float32),
            jax.ShapeDtypeStruct((n, d), jnp.float32),
        ],
    )(aggp2, degp0, degp1, h1, Wl2.T, bl2.reshape(1, d), Wr2.T,
      g2.reshape(1, d), b2.reshape(1, d), Wd.T, bd.reshape(1, d_out))

    return (logits, h2)


# speedup vs baseline: 1.8051x; 1.0468x over previous
import functools

import jax
import jax.numpy as jnp
from jax import lax
from jax.experimental import pallas as pl
from jax.experimental.pallas import tpu as pltpu
from jax.experimental.pallas import tpu_sc as plsc

_CHUNK = 80


def _round_up(v, m):
    return (v + m - 1) // m * m


def _make_sc_agg(n_rows, n_edges, dim, with_deg, chunk):
    info = plsc.get_sparse_core_info()
    nc, ns = info.num_cores, info.num_subcores
    nw = nc * ns
    nchunks = n_edges // chunk
    steps = nchunks // nw
    rps = n_rows // ns

    mesh = plsc.VectorSubcoreMesh(core_axis_name="c", subcore_axis_name="s")

    out_type = jax.ShapeDtypeStruct((nc * n_rows, dim), jnp.float32)
    scratch = [
        pltpu.VMEM_SHARED((n_rows, dim), jnp.float32),
        pltpu.VMEM((2, 8, chunk), jnp.int32),
        pltpu.VMEM((2, chunk, dim), jnp.float32),
        pltpu.SemaphoreType.DMA,
        pltpu.SemaphoreType.DMA,
    ]
    if with_deg:
        scratch.append(pltpu.VMEM((2, chunk, dim), jnp.float32))

    def make_body(feat, idxpack, onehot, zeros_f, out_agg,
                  agg_sh, idx_v, rows_v, sem0, sem1, pay_v):
        sems = (sem0, sem1)
        c = lax.axis_index("c")
        s = lax.axis_index("s")
        wid = s * nc + c
        r0 = s * rps
        pltpu.sync_copy(zeros_f.at[pl.ds(r0, rps)],
                        agg_sh.at[pl.ds(r0, rps)])
        plsc.subcore_barrier()

        base = wid * steps

        def load(i, b):
            pltpu.sync_copy(idxpack.at[base + i], idx_v.at[b])
            pltpu.async_copy(feat.at[idx_v.at[b, 0]], rows_v.at[b], sems[b])
            if with_deg:
                pltpu.async_copy(onehot.at[idx_v.at[b, 2]], pay_v.at[b],
                                 sems[b])

        def drain_scatter(b):
            pltpu.make_async_copy(feat.at[idx_v.at[b, 0]], rows_v.at[b],
                                  sems[b]).wait()
            if with_deg:
                pltpu.make_async_copy(onehot.at[idx_v.at[b, 2]], pay_v.at[b],
                                      sems[b]).wait()
            pltpu.sync_copy(rows_v.at[b], agg_sh.at[idx_v.at[b, 1]],
                            add=True)
            if with_deg:
                pltpu.sync_copy(pay_v.at[b], agg_sh.at[idx_v.at[b, 3]],
                                add=True)

        load(0, 0)
        if steps > 1:
            load(1, 1)

        def pair(p, carry):
            i0 = 2 * p

            drain_scatter(0)

            @pl.when(i0 + 2 < steps)
            def _():
                load(i0 + 2, 0)

            drain_scatter(1)

            @pl.when(i0 + 3 < steps)
            def _():
                load(i0 + 3, 1)

            return carry

        lax.fori_loop(0, steps // 2, pair, 0)
        if steps % 2:
            drain_scatter(0)

        plsc.subcore_barrier()
        o0 = c * n_rows + r0
        pltpu.sync_copy(agg_sh.at[pl.ds(r0, rps)],
                        out_agg.at[pl.ds(o0, rps)])

    if with_deg:
        @functools.partial(
            pl.kernel, mesh=mesh, out_type=[out_type], scratch_types=scratch)
        def sc_kernel(feat, idxpack, onehot, zeros_f, out_agg,
                      agg_sh, idx_v, rows_v, sem0, sem1, pay_v):
            make_body(feat, idxpack, onehot, zeros_f, out_agg,
                      agg_sh, idx_v, rows_v, sem0, sem1, pay_v)
    else:
        @functools.partial(
            pl.kernel, mesh=mesh, out_type=[out_type], scratch_types=scratch)
        def sc_kernel(feat, idxpack, zeros_f, out_agg,
                      agg_sh, idx_v, rows_v, sem0, sem1):
            make_body(feat, idxpack, None, zeros_f, out_agg,
                      agg_sh, idx_v, rows_v, sem0, sem1, None)

    return sc_kernel


def _tc_layer1(aggp_ref, degp0_ref, degp1_ref, x_ref, wl_ref, bl_ref,
               wr_ref, g_ref, b_ref, h_ref):
    n = x_ref.shape[0]
    np_ = aggp_ref.shape[0] // 2
    agg = aggp_ref[0:n, :] + aggp_ref[np_:np_ + n, :]
    deg = degp0_ref[0:n, 0:1] + degp1_ref[0:n, 0:1]
    mean = agg / jnp.maximum(deg, 1.0)
    h = (jnp.dot(mean, wl_ref[...], preferred_element_type=jnp.float32)
         + jnp.dot(x_ref[...], wr_ref[...], preferred_element_type=jnp.float32)
         + bl_ref[...])
    mu = jnp.mean(h, axis=0, keepdims=True)
    var = jnp.mean((h - mu) * (h - mu), axis=0, keepdims=True)
    hn = (h - mu) * lax.rsqrt(var + 1e-5) * g_ref[...] + b_ref[...]
    h_ref[...] = jnp.maximum(hn, 0.0)


def _tc_layer2(aggp_ref, degp0_ref, degp1_ref, x_ref, wl_ref, bl_ref,
               wr_ref, g_ref, b_ref, wd_ref, bd_ref, logits_ref, h_ref):
    n = x_ref.shape[0]
    np_ = aggp_ref.shape[0] // 2
    agg = aggp_ref[0:n, :] + aggp_ref[np_:np_ + n, :]
    deg = degp0_ref[0:n, 0:1] + degp1_ref[0:n, 0:1]
    mean = agg / jnp.maximum(deg, 1.0)
    h = (jnp.dot(mean, wl_ref[...], preferred_element_type=jnp.float32)
         + jnp.dot(x_ref[...], wr_ref[...], preferred_element_type=jnp.float32)
         + bl_ref[...])
    mu = jnp.mean(h, axis=0, keepdims=True)
    var = jnp.mean((h - mu) * (h - mu), axis=0, keepdims=True)
    hn = (h - mu) * lax.rsqrt(var + 1e-5) * g_ref[...] + b_ref[...]
    h2 = jnp.maximum(hn, 0.0)
    h_ref[...] = h2
    logits_ref[...] = (
        jnp.dot(h2, wd_ref[...], preferred_element_type=jnp.float32)
        + bd_ref[...])


def kernel(x, edge_index, Wl1, bl1, Wr1, g1, b1, Wl2, bl2, Wr2, g2, b2,
           Wd, bd):
    n, d = x.shape
    e = edge_index.shape[1]
    d_out = Wd.shape[0]

    n_pad = _round_up(n, 16 * 8)
    n_deg = n_pad // 8
    n_l1 = _round_up(n_pad + n_deg, 16 * 8)
    c1, c2 = 40, _CHUNK

    src = edge_index[0]
    dst = edge_index[1]
    streams = jnp.stack(
        [src, dst, dst % 8, n_pad + dst // 8], axis=0)

    def pack(chunk):
        nch = e // chunk
        p = streams.reshape(4, nch, chunk).transpose(1, 0, 2)
        return jnp.concatenate(
            [p, jnp.zeros((nch, 4, chunk), jnp.int32)], axis=1)

    idxpack1 = pack(c1)
    idxpack2 = pack(c2)
    onehot = (jnp.arange(d, dtype=jnp.int32)[None, :] // 16
              == jnp.arange(8, dtype=jnp.int32)[:, None]).astype(jnp.float32)
    zeros_l1 = jnp.zeros((n_l1, d), jnp.float32)
    zeros_l2 = jnp.zeros((n_pad, d), jnp.float32)

    sc_l1 = _make_sc_agg(n_l1, e, d, with_deg=True, chunk=c1)
    sc_l2 = _make_sc_agg(n_pad, e, d, with_deg=False, chunk=c2)

    out1, = sc_l1(x, idxpack1, onehot, zeros_l1)
    degp0 = out1[n_pad:n_pad + n_deg].reshape(n_pad, 16)
    degp1 = out1[n_l1 + n_pad:n_l1 + n_pad + n_deg].reshape(n_pad, 16)

    h1 = pl.pallas_call(
        _tc_layer1,
        out_shape=jax.ShapeDtypeStruct((n, d), jnp.float32),
    )(out1, degp0, degp1, x, Wl1.T, bl1.reshape(1, d), Wr1.T,
      g1.reshape(1, d), b1.reshape(1, d))

    aggp2, = sc_l2(h1, idxpack2, zeros_l2)

    logits, h2 = pl.pallas_call(
        _tc_layer2,
        out_shape=[
            jax.ShapeDtypeStruct((n, d_out), jnp.float32),
            jax.ShapeDtypeStruct((n, d), jnp.float32),
        ],
    )(aggp2, degp0, degp1, h1, Wl2.T, bl2.reshape(1, d), Wr2.T,
      g2.reshape(1, d), b2.reshape(1, d), Wd.T, bd.reshape(1, d_out))

    return (logits, h2)


# speedup vs baseline: 5.4088x; 2.9964x over previous
import functools

import jax
import jax.numpy as jnp
from jax import lax
from jax.experimental import pallas as pl
from jax.experimental.pallas import tpu as pltpu
from jax.experimental.pallas import tpu_sc as plsc

_CHUNK = 80


def _round_up(v, m):
    return (v + m - 1) // m * m


def _make_sc_agg(n_rows, n_edges, dim, with_deg, chunk):
    info = plsc.get_sparse_core_info()
    nc, ns = info.num_cores, info.num_subcores
    nw = nc * ns
    nchunks = n_edges // chunk
    steps = nchunks // nw
    rps = n_rows // ns

    mesh = plsc.VectorSubcoreMesh(core_axis_name="c", subcore_axis_name="s")

    out_type = jax.ShapeDtypeStruct((nc * n_rows, dim), jnp.float32)
    scratch = [
        pltpu.VMEM_SHARED((n_rows, dim), jnp.float32),
        pltpu.VMEM((2, 8, chunk), jnp.int32),
        pltpu.VMEM((2, chunk, dim), jnp.float32),
        pltpu.SemaphoreType.DMA,
        pltpu.SemaphoreType.DMA,
    ]

    def make_body(feat, idxpack, onehot, zeros_f, out_agg,
                  agg_sh, idx_v, rows_v, sem0, sem1):
        sems = (sem0, sem1)
        c = lax.axis_index("c")
        s = lax.axis_index("s")
        wid = s * nc + c
        r0 = s * rps
        pltpu.sync_copy(zeros_f.at[pl.ds(r0, rps)],
                        agg_sh.at[pl.ds(r0, rps)])
        plsc.subcore_barrier()

        base = wid * steps

        def loop_phase(table, gi, si):
            def load(i, b):
                pltpu.sync_copy(idxpack.at[base + i], idx_v.at[b])
                pltpu.async_copy(table.at[idx_v.at[b, gi]], rows_v.at[b],
                                 sems[b])

            def drain_scatter(b):
                pltpu.make_async_copy(table.at[idx_v.at[b, gi]],
                                      rows_v.at[b], sems[b]).wait()
                pltpu.sync_copy(rows_v.at[b], agg_sh.at[idx_v.at[b, si]],
                                add=True)

            load(0, 0)
            if steps > 1:
                load(1, 1)

            def pair(p, carry):
                i0 = 2 * p

                drain_scatter(0)

                @pl.when(i0 + 2 < steps)
                def _():
                    load(i0 + 2, 0)

                drain_scatter(1)

                @pl.when(i0 + 3 < steps)
                def _():
                    load(i0 + 3, 1)

                return carry

            lax.fori_loop(0, steps // 2, pair, 0)
            if steps % 2:
                drain_scatter(0)

        loop_phase(feat, 0, 1)
        if with_deg:
            loop_phase(onehot, 2, 3)

        plsc.subcore_barrier()
        o0 = c * n_rows + r0
        pltpu.sync_copy(agg_sh.at[pl.ds(r0, rps)],
                        out_agg.at[pl.ds(o0, rps)])

    if with_deg:
        @functools.partial(
            pl.kernel, mesh=mesh, out_type=[out_type], scratch_types=scratch)
        def sc_kernel(feat, idxpack, onehot, zeros_f, out_agg,
                      agg_sh, idx_v, rows_v, sem0, sem1):
            make_body(feat, idxpack, onehot, zeros_f, out_agg,
                      agg_sh, idx_v, rows_v, sem0, sem1)
    else:
        @functools.partial(
            pl.kernel, mesh=mesh, out_type=[out_type], scratch_types=scratch)
        def sc_kernel(feat, idxpack, zeros_f, out_agg,
                      agg_sh, idx_v, rows_v, sem0, sem1):
            make_body(feat, idxpack, None, zeros_f, out_agg,
                      agg_sh, idx_v, rows_v, sem0, sem1)

    return sc_kernel


def _tc_layer1(aggp_ref, degp0_ref, degp1_ref, x_ref, wl_ref, bl_ref,
               wr_ref, g_ref, b_ref, h_ref):
    n = x_ref.shape[0]
    np_ = aggp_ref.shape[0] // 2
    agg = aggp_ref[0:n, :] + aggp_ref[np_:np_ + n, :]
    deg = degp0_ref[0:n, 0:1] + degp1_ref[0:n, 0:1]
    mean = agg / jnp.maximum(deg, 1.0)
    h = (jnp.dot(mean, wl_ref[...], preferred_element_type=jnp.float32)
         + jnp.dot(x_ref[...], wr_ref[...], preferred_element_type=jnp.float32)
         + bl_ref[...])
    mu = jnp.mean(h, axis=0, keepdims=True)
    var = jnp.mean((h - mu) * (h - mu), axis=0, keepdims=True)
    hn = (h - mu) * lax.rsqrt(var + 1e-5) * g_ref[...] + b_ref[...]
    h_ref[...] = jnp.maximum(hn, 0.0)


def _tc_layer2(aggp_ref, degp0_ref, degp1_ref, x_ref, wl_ref, bl_ref,
               wr_ref, g_ref, b_ref, wd_ref, bd_ref, logits_ref, h_ref):
    n = x_ref.shape[0]
    np_ = aggp_ref.shape[0] // 2
    agg = aggp_ref[0:n, :] + aggp_ref[np_:np_ + n, :]
    deg = degp0_ref[0:n, 0:1] + degp1_ref[0:n, 0:1]
    mean = agg / jnp.maximum(deg, 1.0)
    h = (jnp.dot(mean, wl_ref[...], preferred_element_type=jnp.float32)
         + jnp.dot(x_ref[...], wr_ref[...], preferred_element_type=jnp.float32)
         + bl_ref[...])
    mu = jnp.mean(h, axis=0, keepdims=True)
    var = jnp.mean((h - mu) * (h - mu), axis=0, keepdims=True)
    hn = (h - mu) * lax.rsqrt(var + 1e-5) * g_ref[...] + b_ref[...]
    h2 = jnp.maximum(hn, 0.0)
    h_ref[...] = h2
    logits_ref[...] = (
        jnp.dot(h2, wd_ref[...], preferred_element_type=jnp.float32)
        + bd_ref[...])


def kernel(x, edge_index, Wl1, bl1, Wr1, g1, b1, Wl2, bl2, Wr2, g2, b2,
           Wd, bd):
    n, d = x.shape
    e = edge_index.shape[1]
    d_out = Wd.shape[0]

    n_pad = _round_up(n, 16 * 8)
    n_deg = n_pad // 8
    n_l1 = _round_up(n_pad + n_deg, 16 * 8)
    nchunks = e // _CHUNK

    src = edge_index[0]
    dst = edge_index[1]
    rep = 64
    onehot = (jnp.arange(d, dtype=jnp.int32)[None, :] // 16
              == jnp.arange(8, dtype=jnp.int32)[:, None]).astype(jnp.float32)
    onehot = jnp.tile(onehot, (rep, 1))
    replica = (jnp.arange(e, dtype=jnp.int32) // _CHUNK) % rep
    streams = jnp.stack(
        [src, dst, dst % 8 + 8 * replica, n_pad + dst // 8], axis=0)
    idxpack = streams.reshape(4, nchunks, _CHUNK).transpose(1, 0, 2)
    idxpack = jnp.concatenate(
        [idxpack, jnp.zeros((nchunks, 4, _CHUNK), jnp.int32)], axis=1)
    zeros_l1 = jnp.zeros((n_l1, d), jnp.float32)
    zeros_l2 = jnp.zeros((n_pad, d), jnp.float32)

    sc_l1 = _make_sc_agg(n_l1, e, d, with_deg=True, chunk=_CHUNK)
    sc_l2 = _make_sc_agg(n_pad, e, d, with_deg=False, chunk=_CHUNK)

    out1, = sc_l1(x, idxpack, onehot, zeros_l1)
    degp0 = out1[n_pad:n_pad + n_deg].reshape(n_pad, 16)
    degp1 = out1[n_l1 + n_pad:n_l1 + n_pad + n_deg].reshape(n_pad, 16)

    h1 = pl.pallas_call(
        _tc_layer1,
        out_shape=jax.ShapeDtypeStruct((n, d), jnp.float32),
    )(out1, degp0, degp1, x, Wl1.T, bl1.reshape(1, d), Wr1.T,
      g1.reshape(1, d), b1.reshape(1, d))

    aggp2, = sc_l2(h1, idxpack, zeros_l2)

    logits, h2 = pl.pallas_call(
        _tc_layer2,
        out_shape=[
            jax.ShapeDtypeStruct((n, d_out), jnp.float32),
            jax.ShapeDtypeStruct((n, d), jnp.float32),
        ],
    )(aggp2, degp0, degp1, h1, Wl2.T, bl2.reshape(1, d), Wr2.T,
      g2.reshape(1, d), b2.reshape(1, d), Wd.T, bd.reshape(1, d_out))

    return (logits, h2)
